# baseline (speedup 1.0000x reference)
import jax
import jax.numpy as jnp
from jax import lax
from jax.experimental import pallas as pl

_B = 4096
_NV = 7
_NM = 3
_NVM = _NV * _NM
_R = 2187
_TB = 512


def _sel(x, v, d):
    return jnp.where(
        d == 0,
        x[:, 3 * v : 3 * v + 1],
        jnp.where(d == 1, x[:, 3 * v + 1 : 3 * v + 2], x[:, 3 * v + 2 : 3 * v + 3]),
    )


def _body(x_ref, out_ref):
    x = x_ref[...]
    c81 = lax.broadcasted_iota(jnp.int32, (1, 81), 1)
    e1 = None
    for v in range(4):
        d = (c81 // 3 ** (3 - v)) % 3
        s = _sel(x, v, d)
        e1 = s if e1 is None else e1 * s
    c27 = lax.broadcasted_iota(jnp.int32, (1, 27), 1)
    e2 = None
    for k, v in enumerate(range(4, 7)):
        d = (c27 // 3 ** (2 - k)) % 3
        s = _sel(x, v, d)
        e2 = s if e2 is None else e2 * s
    rep = jnp.repeat(e1, 27, axis=1)
    til = jnp.repeat(e2[:, None, :], 81, axis=1).reshape(_TB, _R)
    out_ref[...] = rep * til


def kernel(x, mf_indices):
    del mf_indices
    xf = x.reshape(_B, _NVM)
    grid = (_B // _TB,)
    return pl.pallas_call(
        _body,
        grid=grid,
        in_specs=[pl.BlockSpec((_TB, _NVM), lambda i: (i, 0))],
        out_specs=pl.BlockSpec((_TB, _R), lambda i: (i, 0)),
        out_shape=jax.ShapeDtypeStruct((_B, _R), jnp.float32),
    )(xf)


# speedup vs baseline: 3.5456x; 3.5456x over previous
import jax
import jax.numpy as jnp
from jax import lax
from jax.experimental import pallas as pl

_B = 4096
_NV = 7
_NM = 3
_NVM = _NV * _NM
_R = 2187
_TB = 1024


def _body(x_ref, idx_ref, out_ref):
    lx = jnp.log2(x_ref[...])
    idx7 = idx_ref[0:_NV, :]
    idx21 = jnp.repeat(idx7, _NM, axis=0)
    which = lax.broadcasted_iota(jnp.int32, (_NVM, _R), 0) % _NM
    m = (idx21 == which).astype(jnp.float32)
    s = jnp.dot(lx, m, preferred_element_type=jnp.float32)
    out_ref[...] = jnp.exp2(s)


def kernel(x, mf_indices):
    xf = x.reshape(_B, _NVM)
    idx_t = jnp.pad(mf_indices.T, ((0, 1), (0, 0)))
    grid = (_B // _TB,)
    return pl.pallas_call(
        _body,
        grid=grid,
        in_specs=[
            pl.BlockSpec((_TB, _NVM), lambda i: (i, 0)),
            pl.BlockSpec((8, _R), lambda i: (0, 0)),
        ],
        out_specs=pl.BlockSpec((_TB, _R), lambda i: (i, 0)),
        out_shape=jax.ShapeDtypeStruct((_B, _R), jnp.float32),
    )(xf, idx_t)
